# initial kernel scaffold (unmeasured)
import jax
import jax.numpy as jnp
from jax import lax
from jax.experimental import pallas as pl
from jax.experimental.pallas import tpu as pltpu

N_DEV = 16
PACK = 128


def kernel(Q, K, V):
    b, sq, h, d = Q.shape
    kv = K.shape[1]
    bh = b * h

    def body(q_ref, k_ref, v_ref, out_ref, gather_ref, send_sems, recv_sems):
        my_pos = lax.axis_index("i")
        scale = d ** -0.5

        for bb in range(b):
            for hh in range(h):
                row = bb * h + hh
                q_vec = q_ref[bb, :, hh, :]
                k_bh = k_ref[bb, :, hh, :]
                v_bh = v_ref[bb, :, hh, :]
                s = lax.dot_general(
                    q_vec, k_bh,
                    dimension_numbers=(((1,), (1,)), ((), ())),
                    preferred_element_type=jnp.float32,
                ) * scale
                m = jnp.max(s, axis=1, keepdims=True)
                p = jnp.exp(s - m)
                l = jnp.sum(p, axis=1, keepdims=True)
                o = lax.dot_general(
                    p, v_bh,
                    dimension_numbers=(((1,), (0,)), ((), ())),
                    preferred_element_type=jnp.float32,
                )
                gather_ref[my_pos, pl.ds(row, 1), pl.ds(0, d)] = o
                gather_ref[my_pos, pl.ds(row, 1), pl.ds(d, 1)] = m
                gather_ref[my_pos, pl.ds(row, 1), pl.ds(d + 1, 1)] = l

        sends = []
        for off in range(1, N_DEV):
            dst = lax.rem(my_pos + off, N_DEV)
            rdma = pltpu.make_async_remote_copy(
                src_ref=gather_ref.at[my_pos],
                dst_ref=gather_ref.at[my_pos],
                send_sem=send_sems.at[off],
                recv_sem=recv_sems.at[off],
                device_id=(dst,),
                device_id_type=pl.DeviceIdType.MESH,
            )
            rdma.start()
            sends.append(rdma)
        for rdma in sends:
            rdma.wait_send()
        for rdma in sends:
            rdma.wait_recv()

        gm = gather_ref[0, :, pl.ds(d, 1)]
        for s_idx in range(1, N_DEV):
            gm = jnp.maximum(gm, gather_ref[s_idx, :, pl.ds(d, 1)])
        o_tot = jnp.zeros((bh, d), jnp.float32)
        l_tot = jnp.zeros((bh, 1), jnp.float32)
        for s_idx in range(N_DEV):
            w = jnp.exp(gather_ref[s_idx, :, pl.ds(d, 1)] - gm)
            l_tot = l_tot + w * gather_ref[s_idx, :, pl.ds(d + 1, 1)]
            o_tot = o_tot + w * gather_ref[s_idx, :, pl.ds(0, d)]
        out = o_tot / l_tot
        out_ref[:, 0, :, :] = out.reshape(b, h, d)

    return pl.pallas_call(
        body,
        out_shape=jax.ShapeDtypeStruct((b, sq, h, d), jnp.float32),
        in_specs=[
            pl.BlockSpec(memory_space=pltpu.VMEM),
            pl.BlockSpec(memory_space=pltpu.VMEM),
            pl.BlockSpec(memory_space=pltpu.VMEM),
        ],
        out_specs=pl.BlockSpec(memory_space=pltpu.VMEM),
        scratch_shapes=[
            pltpu.VMEM((N_DEV, b * h, PACK), jnp.float32),
            pltpu.SemaphoreType.DMA((N_DEV,)),
            pltpu.SemaphoreType.DMA((N_DEV,)),
        ],
        compiler_params=pltpu.CompilerParams(collective_id=0),
    )(Q, K, V)


# baseline (device time: 62794 ns/iter reference)
import jax
import jax.numpy as jnp
from jax import lax
from jax.experimental import pallas as pl
from jax.experimental.pallas import tpu as pltpu

N_DEV = 16
PACK = 128


def kernel(Q, K, V):
    b, sq, h, d = Q.shape
    kv = K.shape[1]
    bh = b * h

    def body(q_ref, k_ref, v_ref, out_ref, gather_ref, send_sems, recv_sems):
        my_pos = lax.axis_index("i")
        scale = d ** -0.5

        for bb in range(b):
            for hh in range(h):
                row = bb * h + hh
                q_vec = q_ref[bb, :, hh, :]
                k_bh = k_ref[bb, :, hh, :]
                v_bh = v_ref[bb, :, hh, :]
                s = lax.dot_general(
                    q_vec, k_bh,
                    dimension_numbers=(((1,), (1,)), ((), ())),
                    preferred_element_type=jnp.float32,
                ) * scale
                m = jnp.max(s, axis=1, keepdims=True)
                p = jnp.exp(s - m)
                l = jnp.sum(p, axis=1, keepdims=True)
                o = lax.dot_general(
                    p, v_bh,
                    dimension_numbers=(((1,), (0,)), ((), ())),
                    preferred_element_type=jnp.float32,
                )
                gather_ref[my_pos, pl.ds(row, 1), pl.ds(0, d)] = o
                gather_ref[my_pos, pl.ds(row, 1), pl.ds(d, 1)] = m
                gather_ref[my_pos, pl.ds(row, 1), pl.ds(d + 1, 1)] = l

        sends = []
        for off in range(1, N_DEV):
            dst = lax.rem(my_pos + off, N_DEV)
            rdma = pltpu.make_async_remote_copy(
                src_ref=gather_ref.at[my_pos],
                dst_ref=gather_ref.at[my_pos],
                send_sem=send_sems.at[off],
                recv_sem=recv_sems.at[off],
                device_id=(dst,),
                device_id_type=pl.DeviceIdType.MESH,
            )
            rdma.start()
            sends.append(rdma)
        for rdma in sends:
            rdma.wait_send()
        for rdma in sends:
            rdma.wait_recv()

        gm = gather_ref[0, :, pl.ds(d, 1)]
        for s_idx in range(1, N_DEV):
            gm = jnp.maximum(gm, gather_ref[s_idx, :, pl.ds(d, 1)])
        o_tot = jnp.zeros((bh, d), jnp.float32)
        l_tot = jnp.zeros((bh, 1), jnp.float32)
        for s_idx in range(N_DEV):
            w = jnp.exp(gather_ref[s_idx, :, pl.ds(d, 1)] - gm)
            l_tot = l_tot + w * gather_ref[s_idx, :, pl.ds(d + 1, 1)]
            o_tot = o_tot + w * gather_ref[s_idx, :, pl.ds(0, d)]
        out = o_tot / l_tot
        out_ref[:, 0, :, :] = out.reshape(b, h, d)

    return pl.pallas_call(
        body,
        out_shape=jax.ShapeDtypeStruct((b, sq, h, d), jnp.float32),
        in_specs=[
            pl.BlockSpec(memory_space=pltpu.VMEM),
            pl.BlockSpec(memory_space=pltpu.VMEM),
            pl.BlockSpec(memory_space=pltpu.VMEM),
        ],
        out_specs=pl.BlockSpec(memory_space=pltpu.VMEM),
        scratch_shapes=[
            pltpu.VMEM((N_DEV, b * h, PACK), jnp.float32),
            pltpu.SemaphoreType.DMA((N_DEV,)),
            pltpu.SemaphoreType.DMA((N_DEV,)),
        ],
    )(Q, K, V)


# device time: 32267 ns/iter; 1.9461x vs baseline; 1.9461x over previous
import jax
import jax.numpy as jnp
from jax import lax
from jax.experimental import pallas as pl
from jax.experimental.pallas import tpu as pltpu

N_DEV = 16
PACK = 128


def kernel(Q, K, V):
    b, sq, h, d = Q.shape
    kv = K.shape[1]
    hd = h * d

    Q2 = Q.reshape(b, hd)
    K2 = K.reshape(b, kv, hd)
    V2 = V.reshape(b, kv, hd)

    def body(q_ref, k_ref, v_ref, out_ref, gather_ref, send_sems, recv_sems):
        my_pos = lax.axis_index("i")
        scale = d ** -0.5

        col_h = lax.broadcasted_iota(jnp.int32, (h, hd), 1) // d
        row_h = lax.broadcasted_iota(jnp.int32, (h, hd), 0)
        hmask = col_h == row_h

        for bb in range(b):
            qflat = q_ref[pl.ds(bb, 1), :]
            qmask_t = jnp.where(hmask, qflat, 0.0)
            k_b = k_ref[bb]
            v_b = v_ref[bb]
            s_t = lax.dot_general(
                qmask_t, k_b,
                dimension_numbers=(((1,), (1,)), ((), ())),
                preferred_element_type=jnp.float32,
            ) * scale
            m = jnp.max(s_t, axis=1, keepdims=True)
            p = jnp.exp(s_t - m)
            l = jnp.sum(p, axis=1, keepdims=True)
            o_full = lax.dot_general(
                p, v_b,
                dimension_numbers=(((1,), (0,)), ((), ())),
                preferred_element_type=jnp.float32,
            )
            o = jnp.concatenate(
                [o_full[hh:hh + 1, hh * d:(hh + 1) * d] for hh in range(h)],
                axis=0,
            )
            gather_ref[my_pos, pl.ds(bb * h, h), pl.ds(0, d)] = o
            gather_ref[my_pos, pl.ds(bb * h, h), pl.ds(d, 1)] = m
            gather_ref[my_pos, pl.ds(bb * h, h), pl.ds(d + 1, 1)] = l

        sends = []
        for off in range(1, N_DEV):
            dst = lax.rem(my_pos + off, N_DEV)
            rdma = pltpu.make_async_remote_copy(
                src_ref=gather_ref.at[my_pos],
                dst_ref=gather_ref.at[my_pos],
                send_sem=send_sems.at[off],
                recv_sem=recv_sems.at[off],
                device_id=(dst,),
                device_id_type=pl.DeviceIdType.MESH,
            )
            rdma.start()
            sends.append(rdma)
        for rdma in sends:
            rdma.wait_send()
        for rdma in sends:
            rdma.wait_recv()

        gm = gather_ref[0, :, pl.ds(d, 1)]
        for s_idx in range(1, N_DEV):
            gm = jnp.maximum(gm, gather_ref[s_idx, :, pl.ds(d, 1)])
        o_tot = jnp.zeros((b * h, d), jnp.float32)
        l_tot = jnp.zeros((b * h, 1), jnp.float32)
        for s_idx in range(N_DEV):
            w = jnp.exp(gather_ref[s_idx, :, pl.ds(d, 1)] - gm)
            l_tot = l_tot + w * gather_ref[s_idx, :, pl.ds(d + 1, 1)]
            o_tot = o_tot + w * gather_ref[s_idx, :, pl.ds(0, d)]
        out = o_tot / l_tot
        out_ref[:, 0, :, :] = out.reshape(b, h, d)

    return pl.pallas_call(
        body,
        out_shape=jax.ShapeDtypeStruct((b, sq, h, d), jnp.float32),
        in_specs=[
            pl.BlockSpec(memory_space=pltpu.VMEM),
            pl.BlockSpec(memory_space=pltpu.VMEM),
            pl.BlockSpec(memory_space=pltpu.VMEM),
        ],
        out_specs=pl.BlockSpec(memory_space=pltpu.VMEM),
        scratch_shapes=[
            pltpu.VMEM((N_DEV, b * h, PACK), jnp.float32),
            pltpu.SemaphoreType.DMA((N_DEV,)),
            pltpu.SemaphoreType.DMA((N_DEV,)),
        ],
    )(Q2, K2, V2)
